# Initial kernel scaffold; baseline (speedup 1.0000x reference)
import jax
import jax.numpy as jnp
from jax.experimental import pallas as pl


def kernel(x, edge_index, W1, b1, W2, b2, temp):
    raise NotImplementedError("write your pallas kernel here")


# speedup vs baseline: 14.8642x; 14.8642x over previous
import functools

import jax
import jax.numpy as jnp
from jax import lax
from jax.experimental import pallas as pl
from jax.experimental.pallas import tpu as pltpu
from jax.experimental.pallas import tpu_sc as plsc

_N = 10000
_NP = 10240
_E = 320000
_DIN = 128
_DH = 128
_DOUT = 64
_K = 10

_NC = 2
_NS = 16
_NW = _NC * _NS
_EPW = _E // _NW
_C = 80
_CH = _EPW // _C
_RPT = _NP // _NS
_ZR = 128
_HD = 16

_mesh = plsc.VectorSubcoreMesh(core_axis_name="c", subcore_axis_name="s")
_sc_params = pltpu.CompilerParams(use_tc_tiling_on_sc=False)


def _zero_vmem_2d(ref, rows, cols):
    zk = jnp.zeros((16,), jnp.float32)

    def body(i, carry):
        for c in range(cols // 16):
            ref[i, pl.ds(c * 16, 16)] = zk
        return carry

    lax.fori_loop(0, rows, body, 0)


@functools.partial(
    pl.kernel,
    mesh=_mesh,
    out_type=jax.ShapeDtypeStruct((_NC, _NP, _HD), jnp.float32),
    scratch_types=[
        pltpu.VMEM((_CH, _C), jnp.int32),
        pltpu.VMEM((_C, _HD), jnp.float32),
        pltpu.VMEM((_ZR, _HD), jnp.float32),
        pltpu.VMEM_SHARED((_NP, _HD), jnp.float32),
    ],
    compiler_params=_sc_params,
)
def _hist_sc(col_hbm, out_hbm, colv, onesb, zbuf, acc):
    cid = lax.axis_index("c")
    sid = lax.axis_index("s")
    wid = cid * _NS + sid

    _zero_vmem_2d(zbuf, _ZR, _HD)
    ok = jnp.ones((16,), jnp.float32)

    def setones(i, carry):
        onesb[i, pl.ds(0, 16)] = ok
        return carry

    lax.fori_loop(0, _C, setones, 0)

    for z in range(_RPT // _ZR):
        pltpu.sync_copy(zbuf, acc.at[pl.ds(sid * _RPT + z * _ZR, _ZR)])
    plsc.subcore_barrier()

    pltpu.sync_copy(col_hbm.at[wid], colv)

    def chunk(j, carry):
        pltpu.sync_copy(onesb, acc.at[colv.at[j]], add=True)
        return carry

    lax.fori_loop(0, _CH, chunk, 0)

    plsc.subcore_barrier()
    pltpu.sync_copy(
        acc.at[pl.ds(sid * _RPT, _RPT)],
        out_hbm.at[cid, pl.ds(sid * _RPT, _RPT)],
    )


@functools.partial(
    pl.kernel,
    mesh=_mesh,
    out_type=jax.ShapeDtypeStruct((_NC, _NP, _DOUT), jnp.float32),
    scratch_types=[
        pltpu.VMEM((_CH, _C), jnp.int32),
        pltpu.VMEM((_CH, _C), jnp.int32),
        pltpu.VMEM((_C, _DOUT), jnp.float32),
        pltpu.VMEM((_ZR, _DOUT), jnp.float32),
        pltpu.VMEM_SHARED((_NP, _DOUT), jnp.float32),
        pltpu.SemaphoreType.DMA,
    ],
    compiler_params=_sc_params,
)
def _hop_sc(g_hbm, row_hbm, col_hbm, out_hbm, rowv, colv, buf, zbuf, acc, sem):
    cid = lax.axis_index("c")
    sid = lax.axis_index("s")
    wid = cid * _NS + sid

    _zero_vmem_2d(zbuf, _ZR, _DOUT)
    for z in range(_RPT // _ZR):
        pltpu.sync_copy(zbuf, acc.at[pl.ds(sid * _RPT + z * _ZR, _ZR)])
    plsc.subcore_barrier()

    pltpu.sync_copy(row_hbm.at[wid], rowv)
    pltpu.sync_copy(col_hbm.at[wid], colv)

    def chunk(j, carry):
        pltpu.async_copy(g_hbm.at[rowv.at[j]], buf, sem).wait()
        pltpu.sync_copy(buf, acc.at[colv.at[j]], add=True)
        return carry

    lax.fori_loop(0, _CH, chunk, 0)

    plsc.subcore_barrier()
    pltpu.sync_copy(
        acc.at[pl.ds(sid * _RPT, _RPT)],
        out_hbm.at[cid, pl.ds(sid * _RPT, _RPT)],
    )


def _mlp_body(x_ref, w1_ref, b1_ref, w2_ref, b2_ref, o_ref):
    h = jnp.dot(x_ref[...], w1_ref[...], preferred_element_type=jnp.float32)
    h = jnp.maximum(h + b1_ref[...], 0.0)
    o_ref[...] = (
        jnp.dot(h, w2_ref[...], preferred_element_type=jnp.float32) + b2_ref[...]
    )


_MLP_R = 1024


def _mlp(x, w1, b1, w2, b2):
    return pl.pallas_call(
        _mlp_body,
        grid=(_NP // _MLP_R,),
        in_specs=[
            pl.BlockSpec((_MLP_R, _DIN), lambda i: (i, 0)),
            pl.BlockSpec((_DIN, _DH), lambda i: (0, 0)),
            pl.BlockSpec((1, _DH), lambda i: (0, 0)),
            pl.BlockSpec((_DH, _DOUT), lambda i: (0, 0)),
            pl.BlockSpec((1, _DOUT), lambda i: (0, 0)),
        ],
        out_specs=pl.BlockSpec((_MLP_R, _DOUT), lambda i: (i, 0)),
        out_shape=jax.ShapeDtypeStruct((_NP, _DOUT), jnp.float32),
    )(x, w1, b1.reshape(1, _DH), w2, b2.reshape(1, _DOUT))


def _prep_body(dp_ref, h_ref, t0_ref, dinv_ref, g_ref, hid_ref):
    deg = dp_ref[0] + dp_ref[1] + 1.0
    dinv = lax.rsqrt(deg)[:, 0:1]
    dinvb = jnp.broadcast_to(dinv, (dinv.shape[0], _DOUT))
    h = h_ref[...]
    dinv_ref[...] = dinvb
    g_ref[...] = dinvb * h
    hid_ref[...] = t0_ref[0, 0] * h


_EW_R = 1024


def _prep(dp, h0, t0):
    return pl.pallas_call(
        _prep_body,
        grid=(_NP // _EW_R,),
        in_specs=[
            pl.BlockSpec((_NC, _EW_R, _HD), lambda i: (0, i, 0)),
            pl.BlockSpec((_EW_R, _DOUT), lambda i: (i, 0)),
            pl.BlockSpec((1, 1), lambda i: (0, 0)),
        ],
        out_specs=[
            pl.BlockSpec((_EW_R, _DOUT), lambda i: (i, 0)),
            pl.BlockSpec((_EW_R, _DOUT), lambda i: (i, 0)),
            pl.BlockSpec((_EW_R, _DOUT), lambda i: (i, 0)),
        ],
        out_shape=[
            jax.ShapeDtypeStruct((_NP, _DOUT), jnp.float32),
            jax.ShapeDtypeStruct((_NP, _DOUT), jnp.float32),
            jax.ShapeDtypeStruct((_NP, _DOUT), jnp.float32),
        ],
    )(dp, h0, t0)


def _combine_body(p_ref, g_ref, hid_ref, dinv_ref, t_ref, g_out, hid_out):
    s = dinv_ref[...] * (p_ref[0] + p_ref[1] + g_ref[...])
    hid_out[...] = hid_ref[...] + t_ref[0, 0] * s
    g_out[...] = dinv_ref[...] * s


def _combine(p, g, hid, dinv, t):
    return pl.pallas_call(
        _combine_body,
        grid=(_NP // _EW_R,),
        in_specs=[
            pl.BlockSpec((_NC, _EW_R, _DOUT), lambda i: (0, i, 0)),
            pl.BlockSpec((_EW_R, _DOUT), lambda i: (i, 0)),
            pl.BlockSpec((_EW_R, _DOUT), lambda i: (i, 0)),
            pl.BlockSpec((_EW_R, _DOUT), lambda i: (i, 0)),
            pl.BlockSpec((1, 1), lambda i: (0, 0)),
        ],
        out_specs=[
            pl.BlockSpec((_EW_R, _DOUT), lambda i: (i, 0)),
            pl.BlockSpec((_EW_R, _DOUT), lambda i: (i, 0)),
        ],
        out_shape=[
            jax.ShapeDtypeStruct((_NP, _DOUT), jnp.float32),
            jax.ShapeDtypeStruct((_NP, _DOUT), jnp.float32),
        ],
    )(p, g, hid, dinv, t)


def kernel(x, edge_index, W1, b1, W2, b2, temp):
    row = edge_index[0].reshape(_NW, _CH, _C)
    col = edge_index[1].reshape(_NW, _CH, _C)

    xp = jnp.pad(x, ((0, _NP - _N), (0, 0)))
    h0 = _mlp(xp, W1, b1, W2, b2)
    dp = _hist_sc(col)
    dinv, g, hidden = _prep(dp, h0, temp[0].reshape(1, 1))

    for k in range(_K):
        p = _hop_sc(g, row, col)
        g, hidden = _combine(p, g, hidden, dinv, temp[k + 1].reshape(1, 1))
    return hidden[:_N]


# speedup vs baseline: 26.0630x; 1.7534x over previous
import functools

import jax
import jax.numpy as jnp
from jax import lax
from jax.experimental import pallas as pl
from jax.experimental.pallas import tpu as pltpu
from jax.experimental.pallas import tpu_sc as plsc

_N = 10000
_NP = 10240
_E = 320000
_DIN = 128
_DH = 128
_DOUT = 64
_K = 10

_NC = 2
_NS = 16
_NW = _NC * _NS
_EPW = _E // _NW
_C = 125
_CH = _EPW // _C
_CHP = _CH + 2
_RPT = _NP // _NS
_ZR = 128
_HD = 16

_mesh = plsc.VectorSubcoreMesh(core_axis_name="c", subcore_axis_name="s")
_sc_params = pltpu.CompilerParams(use_tc_tiling_on_sc=False)


def _zero_vmem_2d(ref, rows, cols):
    zk = jnp.zeros((16,), jnp.float32)

    def body(i, carry):
        for c in range(cols // 16):
            ref[i, pl.ds(c * 16, 16)] = zk
        return carry

    lax.fori_loop(0, rows, body, 0)


@functools.partial(
    pl.kernel,
    mesh=_mesh,
    out_type=jax.ShapeDtypeStruct((_NC, _NP, _HD), jnp.float32),
    scratch_types=[
        pltpu.VMEM((_CHP, _C), jnp.int32),
        pltpu.VMEM((_C, _HD), jnp.float32),
        pltpu.VMEM((_ZR, _HD), jnp.float32),
        pltpu.VMEM_SHARED((_NP, _HD), jnp.float32),
    ],
    compiler_params=_sc_params,
)
def _hist_sc(col_hbm, out_hbm, colv, onesb, zbuf, acc):
    cid = lax.axis_index("c")
    sid = lax.axis_index("s")
    wid = cid * _NS + sid

    _zero_vmem_2d(zbuf, _ZR, _HD)
    ok = jnp.ones((16,), jnp.float32)

    def setones(i, carry):
        onesb[i, pl.ds(0, 16)] = ok
        return carry

    lax.fori_loop(0, _C, setones, 0)

    for z in range(_RPT // _ZR):
        pltpu.sync_copy(zbuf, acc.at[pl.ds(sid * _RPT + z * _ZR, _ZR)])
    plsc.subcore_barrier()

    pltpu.sync_copy(col_hbm.at[wid], colv)

    def chunk(j, carry):
        pltpu.sync_copy(onesb, acc.at[colv.at[j]], add=True)
        return carry

    lax.fori_loop(0, _CH, chunk, 0)

    plsc.subcore_barrier()
    pltpu.sync_copy(
        acc.at[pl.ds(sid * _RPT, _RPT)],
        out_hbm.at[cid, pl.ds(sid * _RPT, _RPT)],
    )


@functools.partial(
    pl.kernel,
    mesh=_mesh,
    out_type=jax.ShapeDtypeStruct((_NC, _NP, _DOUT), jnp.float32),
    scratch_types=[
        pltpu.VMEM((_CHP, _C), jnp.int32),
        pltpu.VMEM((_CHP, _C), jnp.int32),
        pltpu.VMEM((_C, _DOUT), jnp.float32),
        pltpu.VMEM((_C, _DOUT), jnp.float32),
        pltpu.VMEM((_ZR, _DOUT), jnp.float32),
        pltpu.VMEM_SHARED((_NP, _DOUT), jnp.float32),
        pltpu.SemaphoreType.DMA,
        pltpu.SemaphoreType.DMA,
        pltpu.SemaphoreType.DMA,
    ],
    compiler_params=_sc_params,
)
def _hop_sc(g_hbm, row_hbm, col_hbm, out_hbm, rowv, colv, buf0, buf1, zbuf,
            acc, sem0, sem1, zsem):
    cid = lax.axis_index("c")
    sid = lax.axis_index("s")
    wid = cid * _NS + sid
    bufs = (buf0, buf1)
    sems = (sem0, sem1)

    _zero_vmem_2d(zbuf, _ZR, _DOUT)
    for z in range(_RPT // _ZR):
        pltpu.async_copy(zbuf, acc.at[pl.ds(sid * _RPT + z * _ZR, _ZR)], zsem)
    pltpu.sync_copy(row_hbm.at[wid], rowv)
    pltpu.sync_copy(col_hbm.at[wid], colv)
    for z in range(_RPT // _ZR):
        pltpu.make_async_copy(zbuf, acc.at[pl.ds(sid * _RPT, _ZR)], zsem).wait()
    plsc.subcore_barrier()

    for b in range(2):
        pltpu.async_copy(g_hbm.at[rowv.at[b]], bufs[b], sems[b])

    def pair(i, carry):
        for b in range(2):
            j = 2 * i + b
            pltpu.make_async_copy(g_hbm.at[rowv.at[j]], bufs[b], sems[b]).wait()
            pltpu.sync_copy(bufs[b], acc.at[colv.at[j]], add=True)
            pltpu.async_copy(g_hbm.at[rowv.at[j + 2]], bufs[b], sems[b])
        return carry

    lax.fori_loop(0, _CH // 2, pair, 0)
    for b in range(2):
        pltpu.make_async_copy(g_hbm.at[rowv.at[0]], bufs[b], sems[b]).wait()

    plsc.subcore_barrier()
    pltpu.sync_copy(
        acc.at[pl.ds(sid * _RPT, _RPT)],
        out_hbm.at[cid, pl.ds(sid * _RPT, _RPT)],
    )


def _mlp_body(x_ref, w1_ref, b1_ref, w2_ref, b2_ref, o_ref):
    h = jnp.dot(x_ref[...], w1_ref[...], preferred_element_type=jnp.float32)
    h = jnp.maximum(h + b1_ref[...], 0.0)
    o_ref[...] = (
        jnp.dot(h, w2_ref[...], preferred_element_type=jnp.float32) + b2_ref[...]
    )


_MLP_R = 1024


def _mlp(x, w1, b1, w2, b2):
    return pl.pallas_call(
        _mlp_body,
        grid=(_NP // _MLP_R,),
        in_specs=[
            pl.BlockSpec((_MLP_R, _DIN), lambda i: (i, 0)),
            pl.BlockSpec((_DIN, _DH), lambda i: (0, 0)),
            pl.BlockSpec((1, _DH), lambda i: (0, 0)),
            pl.BlockSpec((_DH, _DOUT), lambda i: (0, 0)),
            pl.BlockSpec((1, _DOUT), lambda i: (0, 0)),
        ],
        out_specs=pl.BlockSpec((_MLP_R, _DOUT), lambda i: (i, 0)),
        out_shape=jax.ShapeDtypeStruct((_NP, _DOUT), jnp.float32),
    )(x, w1, b1.reshape(1, _DH), w2, b2.reshape(1, _DOUT))


def _prep_body(dp_ref, h_ref, t0_ref, dinv_ref, g_ref, hid_ref):
    deg = dp_ref[0] + dp_ref[1] + 1.0
    dinv = lax.rsqrt(deg)[:, 0:1]
    dinvb = jnp.broadcast_to(dinv, (dinv.shape[0], _DOUT))
    h = h_ref[...]
    dinv_ref[...] = dinvb
    g_ref[...] = dinvb * h
    hid_ref[...] = t0_ref[0, 0] * h


_EW_R = 1024


def _prep(dp, h0, t0):
    return pl.pallas_call(
        _prep_body,
        grid=(_NP // _EW_R,),
        in_specs=[
            pl.BlockSpec((_NC, _EW_R, _HD), lambda i: (0, i, 0)),
            pl.BlockSpec((_EW_R, _DOUT), lambda i: (i, 0)),
            pl.BlockSpec((1, 1), lambda i: (0, 0)),
        ],
        out_specs=[
            pl.BlockSpec((_EW_R, _DOUT), lambda i: (i, 0)),
            pl.BlockSpec((_EW_R, _DOUT), lambda i: (i, 0)),
            pl.BlockSpec((_EW_R, _DOUT), lambda i: (i, 0)),
        ],
        out_shape=[
            jax.ShapeDtypeStruct((_NP, _DOUT), jnp.float32),
            jax.ShapeDtypeStruct((_NP, _DOUT), jnp.float32),
            jax.ShapeDtypeStruct((_NP, _DOUT), jnp.float32),
        ],
    )(dp, h0, t0)


def _combine_body(p_ref, g_ref, hid_ref, dinv_ref, t_ref, g_out, hid_out):
    s = dinv_ref[...] * (p_ref[0] + p_ref[1] + g_ref[...])
    hid_out[...] = hid_ref[...] + t_ref[0, 0] * s
    g_out[...] = dinv_ref[...] * s


def _combine(p, g, hid, dinv, t):
    return pl.pallas_call(
        _combine_body,
        grid=(_NP // _EW_R,),
        in_specs=[
            pl.BlockSpec((_NC, _EW_R, _DOUT), lambda i: (0, i, 0)),
            pl.BlockSpec((_EW_R, _DOUT), lambda i: (i, 0)),
            pl.BlockSpec((_EW_R, _DOUT), lambda i: (i, 0)),
            pl.BlockSpec((_EW_R, _DOUT), lambda i: (i, 0)),
            pl.BlockSpec((1, 1), lambda i: (0, 0)),
        ],
        out_specs=[
            pl.BlockSpec((_EW_R, _DOUT), lambda i: (i, 0)),
            pl.BlockSpec((_EW_R, _DOUT), lambda i: (i, 0)),
        ],
        out_shape=[
            jax.ShapeDtypeStruct((_NP, _DOUT), jnp.float32),
            jax.ShapeDtypeStruct((_NP, _DOUT), jnp.float32),
        ],
    )(p, g, hid, dinv, t)


def kernel(x, edge_index, W1, b1, W2, b2, temp):
    r3 = edge_index[0].reshape(_NW, _CH, _C)
    c3 = edge_index[1].reshape(_NW, _CH, _C)
    row = jnp.concatenate([r3, r3[:, :2]], axis=1)
    col = jnp.concatenate([c3, c3[:, :2]], axis=1)

    xp = jnp.pad(x, ((0, _NP - _N), (0, 0)))
    h0 = _mlp(xp, W1, b1, W2, b2)
    dp = _hist_sc(col)
    dinv, g, hidden = _prep(dp, h0, temp[0].reshape(1, 1))

    for k in range(_K):
        p = _hop_sc(g, row, col)
        g, hidden = _combine(p, g, hidden, dinv, temp[k + 1].reshape(1, 1))
    return hidden[:_N]


# speedup vs baseline: 27.1727x; 1.0426x over previous
import functools

import jax
import jax.numpy as jnp
from jax import lax
from jax.experimental import pallas as pl
from jax.experimental.pallas import tpu as pltpu
from jax.experimental.pallas import tpu_sc as plsc

_N = 10000
_NP = 10240
_E = 320000
_DIN = 128
_DH = 128
_DOUT = 64
_K = 10

_NC = 2
_NS = 16
_NW = _NC * _NS
_EPW = _E // _NW
_C = 125
_CH = _EPW // _C
_CHP = _CH + 2
_RPT = _NP // _NS
_ZR = 128
_HD = 16

_mesh = plsc.VectorSubcoreMesh(core_axis_name="c", subcore_axis_name="s")
_sc_params = pltpu.CompilerParams(use_tc_tiling_on_sc=False)


def _zero_vmem_2d(ref, rows, cols):
    zk = jnp.zeros((16,), jnp.float32)

    def body(i, carry):
        for c in range(cols // 16):
            ref[i, pl.ds(c * 16, 16)] = zk
        return carry

    lax.fori_loop(0, rows, body, 0)


@functools.partial(
    pl.kernel,
    mesh=_mesh,
    out_type=jax.ShapeDtypeStruct((_NC, _NP, _HD), jnp.float32),
    scratch_types=[
        pltpu.VMEM((_CHP, _C), jnp.int32),
        pltpu.VMEM((_C, _HD), jnp.float32),
        pltpu.VMEM((_ZR, _HD), jnp.float32),
        pltpu.VMEM_SHARED((_NP, _HD), jnp.float32),
    ],
    compiler_params=_sc_params,
)
def _hist_sc(col_hbm, out_hbm, colv, onesb, zbuf, acc):
    cid = lax.axis_index("c")
    sid = lax.axis_index("s")
    wid = cid * _NS + sid

    _zero_vmem_2d(zbuf, _ZR, _HD)
    ok = jnp.ones((16,), jnp.float32)

    def setones(i, carry):
        onesb[i, pl.ds(0, 16)] = ok
        return carry

    lax.fori_loop(0, _C, setones, 0)

    for z in range(_RPT // _ZR):
        pltpu.sync_copy(zbuf, acc.at[pl.ds(sid * _RPT + z * _ZR, _ZR)])
    plsc.subcore_barrier()

    pltpu.sync_copy(col_hbm.at[wid], colv)

    def chunk(j, carry):
        pltpu.sync_copy(onesb, acc.at[colv.at[j]], add=True)
        return carry

    lax.fori_loop(0, _CH, chunk, 0)

    plsc.subcore_barrier()
    pltpu.sync_copy(
        acc.at[pl.ds(sid * _RPT, _RPT)],
        out_hbm.at[cid, pl.ds(sid * _RPT, _RPT)],
    )


@functools.partial(
    pl.kernel,
    mesh=_mesh,
    out_type=jax.ShapeDtypeStruct((_NC, _NP, _DOUT), jnp.float32),
    scratch_types=[
        pltpu.VMEM((_CHP, _C), jnp.int32),
        pltpu.VMEM((_CHP, _C), jnp.int32),
        pltpu.VMEM((_C, _DOUT), jnp.float32),
        pltpu.VMEM((_C, _DOUT), jnp.float32),
        pltpu.VMEM((_C, _DOUT), jnp.float32),
        pltpu.VMEM((_C, _DOUT), jnp.float32),
        pltpu.VMEM((_ZR, _DOUT), jnp.float32),
        pltpu.VMEM_SHARED((_NP, _DOUT), jnp.float32),
        [pltpu.SemaphoreType.DMA] * 4,
        [pltpu.SemaphoreType.DMA] * 4,
        pltpu.SemaphoreType.DMA,
    ],
    compiler_params=_sc_params,
)
def _hop_sc(g_hbm, row_hbm, col_hbm, out_hbm, rowv, colv, b0, b1, b2, b3,
            zbuf, acc, gsem, ssem, zsem):
    cid = lax.axis_index("c")
    sid = lax.axis_index("s")
    wid = cid * _NS + sid
    bufs = (b0, b1, b2, b3)

    def g_start(j, slot):
        pltpu.async_copy(g_hbm.at[rowv.at[j]], bufs[slot], gsem[slot])

    def g_wait(slot):
        pltpu.make_async_copy(g_hbm.at[rowv.at[0]], bufs[slot], gsem[slot]).wait()

    def s_start(j, slot):
        pltpu.async_copy(bufs[slot], acc.at[colv.at[j]], ssem[slot], add=True)

    def s_wait(slot):
        pltpu.make_async_copy(bufs[slot], acc.at[colv.at[0]], ssem[slot]).wait()

    _zero_vmem_2d(zbuf, _ZR, _DOUT)
    for z in range(_RPT // _ZR):
        pltpu.async_copy(zbuf, acc.at[pl.ds(sid * _RPT + z * _ZR, _ZR)], zsem)
    pltpu.sync_copy(row_hbm.at[wid], rowv)
    pltpu.sync_copy(col_hbm.at[wid], colv)
    for z in range(_RPT // _ZR):
        pltpu.make_async_copy(zbuf, acc.at[pl.ds(sid * _RPT, _ZR)], zsem).wait()
    plsc.subcore_barrier()

    g_start(0, 0)
    g_start(1, 1)
    g_wait(0); s_start(0, 0); g_start(2, 2)
    g_wait(1); s_start(1, 1); g_start(3, 3)
    g_wait(2); s_start(2, 2); s_wait(0); g_start(4, 0)
    g_wait(3); s_start(3, 3); s_wait(1); g_start(5, 1)

    def group(gi, carry):
        base = 4 * gi
        for b in range(4):
            j = base + b
            g_wait(b)
            s_start(j, b)
            s_wait((b + 2) % 4)
            g_start(j + 2, (b + 2) % 4)
        return carry

    lax.fori_loop(1, _CH // 4, group, 0)
    g_wait(0)
    g_wait(1)
    s_wait(2)
    s_wait(3)

    plsc.subcore_barrier()
    pltpu.sync_copy(
        acc.at[pl.ds(sid * _RPT, _RPT)],
        out_hbm.at[cid, pl.ds(sid * _RPT, _RPT)],
    )


def _mlp_body(x_ref, w1_ref, b1_ref, w2_ref, b2_ref, o_ref):
    h = jnp.dot(x_ref[...], w1_ref[...], preferred_element_type=jnp.float32)
    h = jnp.maximum(h + b1_ref[...], 0.0)
    o_ref[...] = (
        jnp.dot(h, w2_ref[...], preferred_element_type=jnp.float32) + b2_ref[...]
    )


_MLP_R = 1024


def _mlp(x, w1, b1, w2, b2):
    return pl.pallas_call(
        _mlp_body,
        grid=(_NP // _MLP_R,),
        in_specs=[
            pl.BlockSpec((_MLP_R, _DIN), lambda i: (i, 0)),
            pl.BlockSpec((_DIN, _DH), lambda i: (0, 0)),
            pl.BlockSpec((1, _DH), lambda i: (0, 0)),
            pl.BlockSpec((_DH, _DOUT), lambda i: (0, 0)),
            pl.BlockSpec((1, _DOUT), lambda i: (0, 0)),
        ],
        out_specs=pl.BlockSpec((_MLP_R, _DOUT), lambda i: (i, 0)),
        out_shape=jax.ShapeDtypeStruct((_NP, _DOUT), jnp.float32),
    )(x, w1, b1.reshape(1, _DH), w2, b2.reshape(1, _DOUT))


def _prep_body(dp_ref, h_ref, t0_ref, dinv_ref, g_ref, hid_ref):
    deg = dp_ref[0] + dp_ref[1] + 1.0
    dinv = lax.rsqrt(deg)[:, 0:1]
    dinvb = jnp.broadcast_to(dinv, (dinv.shape[0], _DOUT))
    h = h_ref[...]
    dinv_ref[...] = dinvb
    g_ref[...] = dinvb * h
    hid_ref[...] = t0_ref[0, 0] * h


_EW_R = 1024


def _prep(dp, h0, t0):
    return pl.pallas_call(
        _prep_body,
        grid=(_NP // _EW_R,),
        in_specs=[
            pl.BlockSpec((_NC, _EW_R, _HD), lambda i: (0, i, 0)),
            pl.BlockSpec((_EW_R, _DOUT), lambda i: (i, 0)),
            pl.BlockSpec((1, 1), lambda i: (0, 0)),
        ],
        out_specs=[
            pl.BlockSpec((_EW_R, _DOUT), lambda i: (i, 0)),
            pl.BlockSpec((_EW_R, _DOUT), lambda i: (i, 0)),
            pl.BlockSpec((_EW_R, _DOUT), lambda i: (i, 0)),
        ],
        out_shape=[
            jax.ShapeDtypeStruct((_NP, _DOUT), jnp.float32),
            jax.ShapeDtypeStruct((_NP, _DOUT), jnp.float32),
            jax.ShapeDtypeStruct((_NP, _DOUT), jnp.float32),
        ],
    )(dp, h0, t0)


def _combine_body(p_ref, g_ref, hid_ref, dinv_ref, t_ref, g_out, hid_out):
    s = dinv_ref[...] * (p_ref[0] + p_ref[1] + g_ref[...])
    hid_out[...] = hid_ref[...] + t_ref[0, 0] * s
    g_out[...] = dinv_ref[...] * s


def _combine(p, g, hid, dinv, t):
    return pl.pallas_call(
        _combine_body,
        grid=(_NP // _EW_R,),
        in_specs=[
            pl.BlockSpec((_NC, _EW_R, _DOUT), lambda i: (0, i, 0)),
            pl.BlockSpec((_EW_R, _DOUT), lambda i: (i, 0)),
            pl.BlockSpec((_EW_R, _DOUT), lambda i: (i, 0)),
            pl.BlockSpec((_EW_R, _DOUT), lambda i: (i, 0)),
            pl.BlockSpec((1, 1), lambda i: (0, 0)),
        ],
        out_specs=[
            pl.BlockSpec((_EW_R, _DOUT), lambda i: (i, 0)),
            pl.BlockSpec((_EW_R, _DOUT), lambda i: (i, 0)),
        ],
        out_shape=[
            jax.ShapeDtypeStruct((_NP, _DOUT), jnp.float32),
            jax.ShapeDtypeStruct((_NP, _DOUT), jnp.float32),
        ],
    )(p, g, hid, dinv, t)


def kernel(x, edge_index, W1, b1, W2, b2, temp):
    r3 = edge_index[0].reshape(_NW, _CH, _C)
    c3 = edge_index[1].reshape(_NW, _CH, _C)
    row = jnp.concatenate([r3, r3[:, :2]], axis=1)
    col = jnp.concatenate([c3, c3[:, :2]], axis=1)

    xp = jnp.pad(x, ((0, _NP - _N), (0, 0)))
    h0 = _mlp(xp, W1, b1, W2, b2)
    dp = _hist_sc(col)
    dinv, g, hidden = _prep(dp, h0, temp[0].reshape(1, 1))

    for k in range(_K):
        p = _hop_sc(g, row, col)
        g, hidden = _combine(p, g, hidden, dinv, temp[k + 1].reshape(1, 1))
    return hidden[:_N]
